# baseline (speedup 1.0000x reference)
import functools

import jax
import jax.numpy as jnp
from jax import lax
from jax.experimental import pallas as pl
from jax.experimental.pallas import tpu as pltpu
from jax.experimental.pallas import tpu_sc as plsc

ROWS = 128
COLS = 8192
TOPK = 256
LANES = 16
NV = COLS // LANES
NC, NS = 2, 16
NW = NC * NS
ROWS_PER_W = ROWS // NW
UNROLL = 4

MIN32 = -2147483648


def _scan_hist(hist, kneed):
  lane = lax.iota(jnp.int32, LANES)

  sums = jnp.zeros((LANES,), jnp.int32)
  for c in range(LANES):
    s = jnp.sum(hist[pl.ds(c * LANES, LANES)])
    sums = jnp.where(lane == c, s, sums)

  sfx = lax.rev(plsc.cumsum(lax.rev(sums, (0,))), (0,))
  condv = sfx >= kneed
  npc = plsc.all_reduce_population_count(condv)
  cstar = jnp.max(npc) - 1
  above_chunks = jnp.sum(jnp.where(lane == cstar, sfx - sums, 0))

  v = hist[pl.ds(cstar * LANES, LANES)]
  sfx2 = lax.rev(plsc.cumsum(lax.rev(v, (0,))), (0,))
  condv2 = (above_chunks + sfx2) >= kneed
  npc2 = plsc.all_reduce_population_count(condv2)
  jstar = jnp.max(npc2) - 1
  above = above_chunks + jnp.sum(jnp.where(lane == jstar, sfx2 - v, 0))
  cnt = jnp.sum(jnp.where(lane == jstar, v, 0))
  bkt = cstar * LANES + jstar
  return bkt, above, cnt


def kernel(x):
  xi = lax.bitcast_convert_type(x, jnp.int32)

  mesh = plsc.VectorSubcoreMesh(
      core_axis_name="c", subcore_axis_name="s",
      num_cores=NC, num_subcores=NS)

  @functools.partial(
      pl.kernel,
      out_type=jax.ShapeDtypeStruct((ROWS, COLS), jnp.int32),
      mesh=mesh,
      compiler_params=pltpu.CompilerParams(needs_layout_passes=False),
      scratch_types=[
          pltpu.VMEM((COLS,), jnp.int32),
          pltpu.VMEM((COLS,), jnp.int32),
          pltpu.VMEM((256,), jnp.int32),
      ],
  )
  def k(x_hbm, out_hbm, buf, cand, hist):
    wid = lax.axis_index("s") * NC + lax.axis_index("c")
    ones = jnp.ones((LANES,), jnp.int32)
    zeros16 = jnp.zeros((LANES,), jnp.int32)
    lane = lax.iota(jnp.int32, LANES)

    for r in range(ROWS_PER_W):
      row_i = wid * ROWS_PER_W + r
      with jax.named_scope("dma_in"):
        pltpu.sync_copy(x_hbm.at[row_i], buf)

      for c in range(256 // LANES):
        hist[pl.ds(c * LANES, LANES)] = zeros16

      def p0(i, _):
        for u in range(UNROLL):
          sl = pl.ds((i * UNROLL + u) * LANES, LANES)
          bits = buf[sl]
          ub = jnp.where(bits >= 0, bits ^ MIN32, ~bits)
          buf[sl] = ub
          idx = lax.shift_right_logical(ub, 24)
          plsc.addupdate_scatter(hist, [idx], ones)
        return 0
      with jax.named_scope("pass0"):
        lax.fori_loop(0, NV // UNROLL, p0, 0)

      bkt0, above0, m1 = _scan_hist(hist, jnp.int32(TOPK))
      kneed = jnp.int32(TOPK) - above0

      def p1(i, off):
        for u in range(UNROLL):
          sl = pl.ds((i * UNROLL + u) * LANES, LANES)
          ub = buf[sl]
          m = lax.shift_right_logical(ub, 24) == bkt0
          plsc.store_compressed(cand.at[pl.ds(off, LANES)], ub, mask=m)
          off = off + jnp.max(plsc.all_reduce_population_count(m))
        return off
      with jax.named_scope("compact"):
        lax.fori_loop(0, NV // UNROLL, p1, jnp.int32(0))

      nv1 = lax.shift_right_logical(m1 + (LANES - 1), 4)

      digits = jnp.int32(0)
      for level in range(3):
        shift = 16 - 8 * level
        for c in range(256 // LANES):
          hist[pl.ds(c * LANES, LANES)] = zeros16

        if level == 0:
          def mb(i, _):
            sl = pl.ds(i * LANES, LANES)
            ub = cand[sl]
            m = (i * LANES + lane) < m1
            idx = lax.shift_right_logical(ub, 16) & 0xFF
            plsc.addupdate_scatter(hist, [idx], ones, mask=m)
            return 0
        else:
          dg = digits

          def mb(i, _, shift=shift, dg=dg, level=level):
            sl = pl.ds(i * LANES, LANES)
            ub = cand[sl]
            hi = lax.shift_right_logical(ub, shift + 8) & (
                0xFF if level == 1 else 0xFFFF)
            m = jnp.logical_and((i * LANES + lane) < m1, hi == dg)
            idx = lax.shift_right_logical(ub, shift) & 0xFF
            plsc.addupdate_scatter(hist, [idx], ones, mask=m)
            return 0
        with jax.named_scope(f"mini{level}"):
          lax.fori_loop(0, nv1, mb, 0)
        with jax.named_scope(f"scan{level}"):
          b, above, _ = _scan_hist(hist, kneed)
        digits = lax.shift_left(digits, 8) | b
        kneed = kneed - above

      ub_thr = lax.shift_left(bkt0, 24) | digits
      st2 = jnp.maximum(ub_thr ^ MIN32, 1)

      def ob(i, _):
        for u in range(UNROLL):
          sl = pl.ds((i * UNROLL + u) * LANES, LANES)
          s = buf[sl] ^ MIN32
          buf[sl] = jnp.where(s >= st2, s, 0)
        return 0
      with jax.named_scope("output"):
        lax.fori_loop(0, NV // UNROLL, ob, 0)

      with jax.named_scope("dma_out"):
        pltpu.sync_copy(buf, out_hbm.at[row_i])

  return lax.bitcast_convert_type(k(xi), jnp.float32)


# speedup vs baseline: 1.0033x; 1.0033x over previous
import functools

import jax
import jax.numpy as jnp
from jax import lax
from jax.experimental import pallas as pl
from jax.experimental.pallas import tpu as pltpu
from jax.experimental.pallas import tpu_sc as plsc

ROWS = 128
COLS = 8192
TOPK = 256
LANES = 16
NV = COLS // LANES
NC, NS = 2, 16
NW = NC * NS
ROWS_PER_W = ROWS // NW
UNROLL = 8

MIN32 = -2147483648


def _scan_hist(hist, kneed):
  lane = lax.iota(jnp.int32, LANES)

  sums = jnp.zeros((LANES,), jnp.int32)
  for c in range(LANES):
    s = jnp.sum(hist[pl.ds(c * LANES, LANES)])
    sums = jnp.where(lane == c, s, sums)

  sfx = lax.rev(plsc.cumsum(lax.rev(sums, (0,))), (0,))
  condv = sfx >= kneed
  npc = plsc.all_reduce_population_count(condv)
  cstar = jnp.max(npc) - 1
  above_chunks = jnp.sum(jnp.where(lane == cstar, sfx - sums, 0))

  v = hist[pl.ds(cstar * LANES, LANES)]
  sfx2 = lax.rev(plsc.cumsum(lax.rev(v, (0,))), (0,))
  condv2 = (above_chunks + sfx2) >= kneed
  npc2 = plsc.all_reduce_population_count(condv2)
  jstar = jnp.max(npc2) - 1
  above = above_chunks + jnp.sum(jnp.where(lane == jstar, sfx2 - v, 0))
  cnt = jnp.sum(jnp.where(lane == jstar, v, 0))
  bkt = cstar * LANES + jstar
  return bkt, above, cnt


def kernel(x):
  xi = lax.bitcast_convert_type(x, jnp.int32)

  mesh = plsc.VectorSubcoreMesh(
      core_axis_name="c", subcore_axis_name="s",
      num_cores=NC, num_subcores=NS)

  @functools.partial(
      pl.kernel,
      out_type=jax.ShapeDtypeStruct((ROWS, COLS), jnp.int32),
      mesh=mesh,
      compiler_params=pltpu.CompilerParams(needs_layout_passes=False),
      scratch_types=[
          pltpu.VMEM((COLS,), jnp.int32),
          pltpu.VMEM((COLS,), jnp.int32),
          pltpu.VMEM((256,), jnp.int32),
      ],
  )
  def k(x_hbm, out_hbm, buf, cand, hist):
    wid = lax.axis_index("s") * NC + lax.axis_index("c")
    ones = jnp.ones((LANES,), jnp.int32)
    zeros16 = jnp.zeros((LANES,), jnp.int32)
    lane = lax.iota(jnp.int32, LANES)

    for r in range(ROWS_PER_W):
      row_i = wid * ROWS_PER_W + r
      with jax.named_scope("dma_in"):
        pltpu.sync_copy(x_hbm.at[row_i], buf)

      for c in range(256 // LANES):
        hist[pl.ds(c * LANES, LANES)] = zeros16

      def p0(i, _):
        for u in range(UNROLL):
          sl = pl.ds((i * UNROLL + u) * LANES, LANES)
          bits = buf[sl]
          ub = jnp.where(bits >= 0, bits ^ MIN32, ~bits)
          buf[sl] = ub
          idx = lax.shift_right_logical(ub, 24)
          plsc.addupdate_scatter(hist, [idx], ones)
        return 0
      with jax.named_scope("pass0"):
        lax.fori_loop(0, NV // UNROLL, p0, 0)

      bkt0, above0, m1 = _scan_hist(hist, jnp.int32(TOPK))
      kneed = jnp.int32(TOPK) - above0

      def p1(i, off):
        for u in range(UNROLL):
          sl = pl.ds((i * UNROLL + u) * LANES, LANES)
          ub = buf[sl]
          m = lax.shift_right_logical(ub, 24) == bkt0
          plsc.store_compressed(cand.at[pl.ds(off, LANES)], ub, mask=m)
          off = off + jnp.max(plsc.all_reduce_population_count(m))
        return off
      with jax.named_scope("compact"):
        lax.fori_loop(0, NV // UNROLL, p1, jnp.int32(0))

      nv1 = lax.shift_right_logical(m1 + (LANES - 1), 4)

      digits = jnp.int32(0)
      for level in range(3):
        shift = 16 - 8 * level
        for c in range(256 // LANES):
          hist[pl.ds(c * LANES, LANES)] = zeros16

        if level == 0:
          def mb(i, _):
            sl = pl.ds(i * LANES, LANES)
            ub = cand[sl]
            m = (i * LANES + lane) < m1
            idx = lax.shift_right_logical(ub, 16) & 0xFF
            plsc.addupdate_scatter(hist, [idx], ones, mask=m)
            return 0
        else:
          dg = digits

          def mb(i, _, shift=shift, dg=dg, level=level):
            sl = pl.ds(i * LANES, LANES)
            ub = cand[sl]
            hi = lax.shift_right_logical(ub, shift + 8) & (
                0xFF if level == 1 else 0xFFFF)
            m = jnp.logical_and((i * LANES + lane) < m1, hi == dg)
            idx = lax.shift_right_logical(ub, shift) & 0xFF
            plsc.addupdate_scatter(hist, [idx], ones, mask=m)
            return 0
        with jax.named_scope(f"mini{level}"):
          lax.fori_loop(0, nv1, mb, 0)
        with jax.named_scope(f"scan{level}"):
          b, above, _ = _scan_hist(hist, kneed)
        digits = lax.shift_left(digits, 8) | b
        kneed = kneed - above

      ub_thr = lax.shift_left(bkt0, 24) | digits
      st2 = jnp.maximum(ub_thr ^ MIN32, 1)

      def ob(i, _):
        for u in range(UNROLL):
          sl = pl.ds((i * UNROLL + u) * LANES, LANES)
          s = buf[sl] ^ MIN32
          buf[sl] = jnp.where(s >= st2, s, 0)
        return 0
      with jax.named_scope("output"):
        lax.fori_loop(0, NV // UNROLL, ob, 0)

      with jax.named_scope("dma_out"):
        pltpu.sync_copy(buf, out_hbm.at[row_i])

  return lax.bitcast_convert_type(k(xi), jnp.float32)
